# baseline (speedup 1.0000x reference)
import functools

import jax
import jax.numpy as jnp
from jax import lax
from jax.experimental import pallas as pl
from jax.experimental.pallas import tpu as pltpu
from jax.experimental.pallas import tpu_sc as plsc

V = 1000000
B = 4096
S = 200
D = 64
NC = 2
NS = 16
NW = NC * NS
ROWS_PER_W = B // NW
CSIZE = (96, 104, 96, 104)
COFF = (0, 96, 200, 296)
NCHUNK = 4
PREP_BLK = 32768


def _tc_dup_table(emb_t):
    def dup_kernel(x_ref, o_ref):
        xt = x_ref[...].T
        o_ref[...] = jnp.concatenate([xt, xt], axis=1)

    return pl.pallas_call(
        dup_kernel,
        grid=(pl.cdiv(V, PREP_BLK),),
        in_specs=[pl.BlockSpec((D, PREP_BLK), lambda i: (0, i))],
        out_specs=pl.BlockSpec((PREP_BLK, 2 * D), lambda i: (i, 0)),
        out_shape=jax.ShapeDtypeStruct((V, 2 * D), jnp.float32),
        compiler_params=pltpu.CompilerParams(
            vmem_limit_bytes=100 * 1024 * 1024
        ),
    )(emb_t)


def _sc_pooled_features(idx_all, table_dup):
    mesh = plsc.VectorSubcoreMesh(
        core_axis_name="c", subcore_axis_name="s", num_cores=NC, num_subcores=NS
    )

    @functools.partial(
        pl.kernel,
        out_type=jax.ShapeDtypeStruct((B, 2 * D), jnp.float32),
        mesh=mesh,
        scratch_types=[
            pltpu.VMEM((ROWS_PER_W * 2 * S,), jnp.int32),
            pltpu.VMEM((CSIZE[0], 2 * D), jnp.float32),
            pltpu.VMEM((CSIZE[1], 2 * D), jnp.float32),
            pltpu.VMEM((CSIZE[2], 2 * D), jnp.float32),
            pltpu.VMEM((CSIZE[3], 2 * D), jnp.float32),
            pltpu.VMEM((ROWS_PER_W, 2 * D), jnp.float32),
            pltpu.SemaphoreType.DMA,
            pltpu.SemaphoreType.DMA,
            pltpu.SemaphoreType.DMA,
            pltpu.SemaphoreType.DMA,
        ],
    )
    def feat_kernel(p_hbm, h_hbm, table_hbm, out_hbm, idx_v, r0, r1, r2, r3,
                    feat_v, s0, s1, s2, s3):
        wid = lax.axis_index("s") * NC + lax.axis_index("c")
        base = wid * ROWS_PER_W
        sems = (s0, s1, s2, s3)
        rows = (r0, r1, r2, r3)

        pltpu.sync_copy(
            p_hbm.at[pl.ds(base * S, ROWS_PER_W * S)],
            idx_v.at[pl.ds(0, ROWS_PER_W * S)],
        )
        pltpu.sync_copy(
            h_hbm.at[pl.ds(base * S, ROWS_PER_W * S)],
            idx_v.at[pl.ds(ROWS_PER_W * S, ROWS_PER_W * S)],
        )

        def fire(row, c):
            half = 0 if c < 2 else ROWS_PER_W * S
            off = COFF[c] if c < 2 else COFF[c] - S
            pltpu.async_copy(
                table_hbm.at[idx_v.at[pl.ds(half + row * S + off, CSIZE[c])]],
                rows[c],
                sems[c],
            )

        def drain(c):
            pltpu.make_async_copy(
                table_hbm.at[pl.ds(0, CSIZE[c])], rows[c], sems[c]
            ).wait()

        def chunk_max(c, accs):
            def body(j, a):
                return tuple(
                    jnp.maximum(a[k], rows[c][j, pl.ds(k * 16, 16)])
                    for k in range(4)
                )

            return lax.fori_loop(0, CSIZE[c], body, accs, unroll=2)

        neg_inf = tuple(jnp.full((16,), -jnp.inf, jnp.float32) for _ in range(4))

        for c in range(NCHUNK):
            fire(0, c)

        def per_row(i, _):
            p_acc, h_acc = neg_inf, neg_inf
            for c in range(NCHUNK):
                drain(c)
                if c < NCHUNK // 2:
                    p_acc = chunk_max(c, p_acc)
                else:
                    h_acc = chunk_max(c, h_acc)

                @pl.when(i < ROWS_PER_W - 1)
                def _fire_next():
                    fire(i + 1, c)

            for k in range(4):
                feat_v[i, pl.ds(k * 16, 16)] = p_acc[k]
                feat_v[i, pl.ds(D + k * 16, 16)] = h_acc[k]
            return _

        lax.fori_loop(0, ROWS_PER_W, per_row, None)
        pltpu.sync_copy(feat_v, out_hbm.at[pl.ds(base, ROWS_PER_W)])

    return feat_kernel(*idx_all, table_dup)


def _tc_head(feat, W, b):
    def head_kernel(x_ref, w_ref, b_ref, o_ref):
        z = jnp.sum(x_ref[...] * w_ref[...], axis=1, keepdims=True)
        o_ref[...] = jax.nn.sigmoid(z + b_ref[0])

    return pl.pallas_call(
        head_kernel,
        in_specs=[
            pl.BlockSpec(memory_space=pltpu.VMEM),
            pl.BlockSpec(memory_space=pltpu.VMEM),
            pl.BlockSpec(memory_space=pltpu.SMEM),
        ],
        out_shape=jax.ShapeDtypeStruct((B, 1), jnp.float32),
    )(feat, W, b)


def kernel(premise, hypothesis, emb_table, W, b):
    idx_all = (
        premise.astype(jnp.int32).reshape(B * S),
        hypothesis.astype(jnp.int32).reshape(B * S),
    )
    table_dup = _tc_dup_table(emb_table.T)
    feat = _sc_pooled_features(idx_all, table_dup)
    return jnp.ravel(_tc_head(feat, W, b))


# speedup vs baseline: 1.0055x; 1.0055x over previous
import functools

import jax
import jax.numpy as jnp
from jax import lax
from jax.experimental import pallas as pl
from jax.experimental.pallas import tpu as pltpu
from jax.experimental.pallas import tpu_sc as plsc

V = 1000000
B = 4096
S = 200
D = 64
NC = 2
NS = 16
NW = NC * NS
ROWS_PER_W = B // NW
CSIZE = (96, 104, 96, 104)
COFF = (0, 96, 200, 296)
NCHUNK = 4
PREP_BLK = 32768


def _tc_dup_table(emb_t):
    def dup_kernel(x_ref, o_ref):
        xt = x_ref[...].T
        o_ref[...] = jnp.concatenate([xt, xt], axis=1)

    return pl.pallas_call(
        dup_kernel,
        grid=(pl.cdiv(V, PREP_BLK),),
        in_specs=[pl.BlockSpec((D, PREP_BLK), lambda i: (0, i))],
        out_specs=pl.BlockSpec((PREP_BLK, 2 * D), lambda i: (i, 0)),
        out_shape=jax.ShapeDtypeStruct((V, 2 * D), jnp.float32),
        compiler_params=pltpu.CompilerParams(
            vmem_limit_bytes=100 * 1024 * 1024
        ),
    )(emb_t)


def _sc_pooled_features(idx_all, table_dup):
    mesh = plsc.VectorSubcoreMesh(
        core_axis_name="c", subcore_axis_name="s", num_cores=NC, num_subcores=NS
    )

    @functools.partial(
        pl.kernel,
        out_type=jax.ShapeDtypeStruct((B, 2 * D), jnp.float32),
        mesh=mesh,
        scratch_types=[
            pltpu.VMEM((ROWS_PER_W * 2 * S,), jnp.int32),
            pltpu.VMEM((CSIZE[0], 2 * D), jnp.float32),
            pltpu.VMEM((CSIZE[1], 2 * D), jnp.float32),
            pltpu.VMEM((CSIZE[2], 2 * D), jnp.float32),
            pltpu.VMEM((CSIZE[3], 2 * D), jnp.float32),
            pltpu.VMEM((ROWS_PER_W, 2 * D), jnp.float32),
            pltpu.SemaphoreType.DMA,
            pltpu.SemaphoreType.DMA,
            pltpu.SemaphoreType.DMA,
            pltpu.SemaphoreType.DMA,
        ],
    )
    def feat_kernel(idx_hbm, table_hbm, out_hbm, idx_v, r0, r1, r2, r3,
                    feat_v, s0, s1, s2, s3):
        wid = lax.axis_index("s") * NC + lax.axis_index("c")
        base = wid * ROWS_PER_W
        sems = (s0, s1, s2, s3)
        rows = (r0, r1, r2, r3)

        pltpu.sync_copy(
            idx_hbm.at[pl.ds(base * 2 * S, ROWS_PER_W * 2 * S)], idx_v
        )

        def fire(row, c):
            pltpu.async_copy(
                table_hbm.at[idx_v.at[pl.ds(row * 2 * S + COFF[c], CSIZE[c])]],
                rows[c],
                sems[c],
            )

        def drain(c):
            pltpu.make_async_copy(
                table_hbm.at[pl.ds(0, CSIZE[c])], rows[c], sems[c]
            ).wait()

        def chunk_max(c, accs):
            def body(j, a):
                return tuple(
                    jnp.maximum(a[k], rows[c][j, pl.ds(k * 16, 16)])
                    for k in range(4)
                )

            return lax.fori_loop(0, CSIZE[c], body, accs, unroll=2)

        neg_inf = tuple(jnp.full((16,), -jnp.inf, jnp.float32) for _ in range(4))

        for c in range(NCHUNK):
            fire(0, c)

        def per_row(i, _):
            p_acc, h_acc = neg_inf, neg_inf
            for c in range(NCHUNK):
                drain(c)
                if c < NCHUNK // 2:
                    p_acc = chunk_max(c, p_acc)
                else:
                    h_acc = chunk_max(c, h_acc)

                @pl.when(i < ROWS_PER_W - 1)
                def _fire_next():
                    fire(i + 1, c)

            for k in range(4):
                feat_v[i, pl.ds(k * 16, 16)] = p_acc[k]
                feat_v[i, pl.ds(D + k * 16, 16)] = h_acc[k]
            return _

        lax.fori_loop(0, ROWS_PER_W, per_row, None)
        pltpu.sync_copy(feat_v, out_hbm.at[pl.ds(base, ROWS_PER_W)])

    return feat_kernel(idx_all, table_dup)


def _tc_head(feat, W, b):
    def head_kernel(x_ref, w_ref, b_ref, o_ref):
        z = jnp.sum(x_ref[...] * w_ref[...], axis=1, keepdims=True)
        o_ref[...] = jax.nn.sigmoid(z + b_ref[0])

    return pl.pallas_call(
        head_kernel,
        in_specs=[
            pl.BlockSpec(memory_space=pltpu.VMEM),
            pl.BlockSpec(memory_space=pltpu.VMEM),
            pl.BlockSpec(memory_space=pltpu.SMEM),
        ],
        out_shape=jax.ShapeDtypeStruct((B, 1), jnp.float32),
    )(feat, W, b)


def kernel(premise, hypothesis, emb_table, W, b):
    idx_all = jnp.concatenate(
        [premise.astype(jnp.int32), hypothesis.astype(jnp.int32)], axis=1
    ).reshape(B * 2 * S)
    table_dup = _tc_dup_table(emb_table.T)
    feat = _sc_pooled_features(idx_all, table_dup)
    return jnp.ravel(_tc_head(feat, W, b))
